# baseline (speedup 1.0000x reference)
import functools
import math

import jax
import jax.numpy as jnp
import numpy as np
from jax import lax
from jax.experimental import pallas as pl
from jax.experimental.pallas import tpu as pltpu
from jax.experimental.pallas import tpu_sc as plsc

F32 = jnp.float32
EPS = 1e-5
N_NODES = 10000
N_EDGES = 160000
N_LG = 320000


def _row_spec(bm, k):
    return pl.BlockSpec((bm, k), lambda i: (i, 0))


def _const_spec(shape):
    nd = len(shape)
    return pl.BlockSpec(shape, lambda i: (0,) * nd)


def _aff_from_stats(st, m, g, b):
    s = st[0]
    q = st[1]
    mean = s / m
    var = q / m - mean * mean
    a = g * lax.rsqrt(var + EPS)
    c = b - mean * a
    return a, c


def _stats_update(st_ref, acc, n):
    @pl.when(pl.program_id(0) == 0)
    def _():
        st_ref[...] = jnp.zeros_like(st_ref)

    upd = jnp.concatenate(
        [jnp.sum(acc, axis=0)[None], jnp.sum(acc * acc, axis=0)[None],
         jnp.zeros((6, n), F32)], axis=0)
    st_ref[...] += upd


def _mm(inputs, bias, m, bm, stats=False):
    n_in = len(inputs)
    n_out = inputs[0][4].shape[1]
    grid = m // bm
    assert grid * bm == m, (m, bm)

    in_specs = []
    args = []
    has_aff = []
    pres = []
    acts = []
    for (x, pre, aff, act, w) in inputs:
        in_specs.append(_row_spec(bm, x.shape[1]))
        args.append(x)
        if aff is not None:
            kk = w.shape[0]
            in_specs += [_const_spec((1, kk)), _const_spec((1, kk))]
            args += [aff[0].reshape(1, kk), aff[1].reshape(1, kk)]
        in_specs.append(_const_spec(w.shape))
        args.append(w)
        has_aff.append(aff is not None)
        pres.append(pre)
        acts.append(act)
    in_specs.append(_const_spec((1, n_out)))
    args.append(bias.reshape(1, n_out))

    def body(*refs):
        i = 0
        acc = None
        for t in range(n_in):
            xb = refs[i][...]
            i += 1
            if pres[t] is not None:
                xb = pres[t](xb)
            if has_aff[t]:
                a = refs[i][...]
                c = refs[i + 1][...]
                i += 2
                xb = xb * a + c
            if acts[t] is not None:
                xb = acts[t](xb)
            w = refs[i][...]
            i += 1
            y = jnp.dot(xb, w, preferred_element_type=F32)
            acc = y if acc is None else acc + y
        acc = acc + refs[i][...]
        i += 1
        refs[i][...] = acc
        i += 1
        if stats:
            _stats_update(refs[i], acc, n_out)

    out_shape = [jax.ShapeDtypeStruct((m, n_out), F32)]
    out_specs = [_row_spec(bm, n_out)]
    if stats:
        out_shape.append(jax.ShapeDtypeStruct((8, n_out), F32))
        out_specs.append(_const_spec((8, n_out)))
    res = pl.pallas_call(
        body, grid=(grid,), in_specs=in_specs, out_specs=out_specs,
        out_shape=out_shape)(*args)
    return res if stats else res[0]


def _mm_pair(x, aff, act, w1, b1, w2, b2, m, bm):
    grid = m // bm
    assert grid * bm == m
    k = x.shape[1]
    n1 = w1.shape[1]
    n2 = w2.shape[1]
    in_specs = [_row_spec(bm, k)]
    args = [x]
    if aff is not None:
        in_specs += [_const_spec((1, k)), _const_spec((1, k))]
        args += [aff[0].reshape(1, k), aff[1].reshape(1, k)]
    in_specs += [_const_spec(w1.shape), _const_spec((1, n1)),
                 _const_spec(w2.shape), _const_spec((1, n2))]
    args += [w1, b1.reshape(1, n1), w2, b2.reshape(1, n2)]

    def body(*refs):
        i = 0
        xb = refs[i][...]
        i += 1
        if aff is not None:
            a = refs[i][...]
            c = refs[i + 1][...]
            i += 2
            xb = xb * a + c
        if act is not None:
            xb = act(xb)
        w1r, b1r, w2r, b2r = refs[i], refs[i + 1], refs[i + 2], refs[i + 3]
        o1, o2 = refs[i + 4], refs[i + 5]
        o1[...] = jnp.dot(xb, w1r[...], preferred_element_type=F32) + b1r[...]
        o2[...] = jnp.dot(xb, w2r[...], preferred_element_type=F32) + b2r[...]

    return pl.pallas_call(
        body, grid=(grid,), in_specs=in_specs,
        out_specs=[_row_spec(bm, n1), _row_spec(bm, n2)],
        out_shape=[jax.ShapeDtypeStruct((m, n1), F32),
                   jax.ShapeDtypeStruct((m, n2), F32)])(*args)


def _gate(mm_arr, a, c, m, bm, split=False):
    grid = m // bm
    half = mm_arr.shape[1] // 2

    def body(m_ref, a_ref, c_ref, *outs):
        n = m_ref[...] * a_ref[...] + c_ref[...]
        hf = n[:, :half]
        hs = n[:, half:]
        g = jax.nn.sigmoid(hf) * jax.nn.softplus(hs)
        if split:
            outs[0][...] = g[:, :half // 2]
            outs[1][...] = g[:, half // 2:]
        else:
            outs[0][...] = g

    if split:
        out_specs = [_row_spec(bm, half // 2), _row_spec(bm, half // 2)]
        out_shape = [jax.ShapeDtypeStruct((m, half // 2), F32),
                     jax.ShapeDtypeStruct((m, half // 2), F32)]
    else:
        out_specs = [_row_spec(bm, half)]
        out_shape = [jax.ShapeDtypeStruct((m, half), F32)]
    res = pl.pallas_call(
        body, grid=(grid,),
        in_specs=[_row_spec(bm, 2 * half), _const_spec((1, 2 * half)),
                  _const_spec((1, 2 * half))],
        out_specs=out_specs, out_shape=out_shape)(
            mm_arr, a.reshape(1, -1), c.reshape(1, -1))
    return res if split else res[0]


def _ew_affine_act(x, a, c, act, m, bm):
    grid = m // bm
    n = x.shape[1]

    def body(x_ref, a_ref, c_ref, o_ref):
        o_ref[...] = act(x_ref[...] * a_ref[...] + c_ref[...])

    return pl.pallas_call(
        body, grid=(grid,),
        in_specs=[_row_spec(bm, n), _const_spec((1, n)), _const_spec((1, n))],
        out_specs=[_row_spec(bm, n)],
        out_shape=[jax.ShapeDtypeStruct((m, n), F32)])(
            x, a.reshape(1, n), c.reshape(1, n))[0]


def _res_update(x, res, a, c, mode, m, bm, stats=False):
    grid = m // bm
    n = x.shape[1]

    def body(x_ref, r_ref, a_ref, c_ref, o_ref, *maybe_st):
        t = x_ref[...] * a_ref[...] + c_ref[...]
        if mode == 'sp_out':
            o = jax.nn.softplus(r_ref[...] + t)
        else:
            o = r_ref[...] + jax.nn.softplus(t)
        o_ref[...] = o
        if stats:
            _stats_update(maybe_st[0], o, n)

    out_shape = [jax.ShapeDtypeStruct((m, n), F32)]
    out_specs = [_row_spec(bm, n)]
    if stats:
        out_shape.append(jax.ShapeDtypeStruct((8, n), F32))
        out_specs.append(_const_spec((8, n)))
    res_ = pl.pallas_call(
        body, grid=(grid,),
        in_specs=[_row_spec(bm, n), _row_spec(bm, n), _const_spec((1, n)),
                  _const_spec((1, n))],
        out_specs=out_specs, out_shape=out_shape)(
            x, res, a.reshape(1, n), c.reshape(1, n))
    return res_ if stats else res_[0]


def _sum2_stats(a0, a1, m, bm):
    grid = m // bm
    n = a0.shape[1]

    def body(x_ref, y_ref, o_ref, st_ref):
        o = x_ref[...] + y_ref[...]
        o_ref[...] = o
        _stats_update(st_ref, o, n)

    return pl.pallas_call(
        body, grid=(grid,),
        in_specs=[_row_spec(bm, n), _row_spec(bm, n)],
        out_specs=[_row_spec(bm, n), _const_spec((8, n))],
        out_shape=[jax.ShapeDtypeStruct((m, n), F32),
                   jax.ShapeDtypeStruct((8, n), F32)])(a0, a1)


def _stats_only(x, m, bm):
    grid = m // bm
    n = x.shape[1]

    def body(x_ref, st_ref):
        _stats_update(st_ref, x_ref[...], n)

    return pl.pallas_call(
        body, grid=(grid,),
        in_specs=[_row_spec(bm, n)],
        out_specs=[_const_spec((8, n))],
        out_shape=[jax.ShapeDtypeStruct((8, n), F32)])(x)[0]


def _stats_only2(xa, xb, m, bm):
    grid = m // bm
    n = xa.shape[1] + xb.shape[1]

    def body(xa_ref, xb_ref, st_ref):
        x = jnp.concatenate([xa_ref[...], xb_ref[...]], axis=1)
        _stats_update(st_ref, x, n)

    return pl.pallas_call(
        body, grid=(grid,),
        in_specs=[_row_spec(bm, xa.shape[1]), _row_spec(bm, xb.shape[1])],
        out_specs=[_const_spec((8, n))],
        out_shape=[jax.ShapeDtypeStruct((8, n), F32)])(xa, xb)[0]


def _res_update2(xa, xb, res, a, c, m, bm):
    grid = m // bm
    n = xa.shape[1] + xb.shape[1]

    def body(xa_ref, xb_ref, r_ref, a_ref, c_ref, o_ref):
        x = jnp.concatenate([xa_ref[...], xb_ref[...]], axis=1)
        o_ref[...] = jax.nn.softplus(r_ref[...] + x * a_ref[...] + c_ref[...])

    return pl.pallas_call(
        body, grid=(grid,),
        in_specs=[_row_spec(bm, xa.shape[1]), _row_spec(bm, xb.shape[1]),
                  _row_spec(bm, n), _const_spec((1, n)), _const_spec((1, n))],
        out_specs=[_row_spec(bm, n)],
        out_shape=[jax.ShapeDtypeStruct((m, n), F32)])(
            xa, xb, res, a.reshape(1, n), c.reshape(1, n))[0]


def _pool_head(xf, a, c, wfc, bfc, m, bm):
    grid = m // bm
    n = xf.shape[1]

    def body(x_ref, a_ref, c_ref, w_ref, b_ref, o_ref):
        @pl.when(pl.program_id(0) == 0)
        def _():
            o_ref[...] = jnp.zeros_like(o_ref)

        hn = jax.nn.relu(x_ref[...] * a_ref[...] + c_ref[...])
        o_ref[...] += jnp.sum(jnp.dot(hn, w_ref[...],
                                      preferred_element_type=F32),
                              axis=0, keepdims=True)

        @pl.when(pl.program_id(0) == grid - 1)
        def _():
            o_ref[...] = o_ref[...] * (1.0 / m) + b_ref[...]

    return pl.pallas_call(
        body, grid=(grid,),
        in_specs=[_row_spec(bm, n), _const_spec((1, n)), _const_spec((1, n)),
                  _const_spec(wfc.shape), _const_spec((1, 1))],
        out_specs=[_const_spec((1, 1))],
        out_shape=[jax.ShapeDtypeStruct((1, 1), F32)])(
            xf, a.reshape(1, n), c.reshape(1, n), wfc, bfc.reshape(1, 1))[0]


_NW = 32


def _sc_gather3(tab_a, tab_b, lin, idx_a, idx_b, idx_l, m):
    ch = 80 if m % (80 * _NW) == 0 else 40
    nch = m // ch
    per = nch // _NW
    assert per * _NW == nch
    mesh = plsc.VectorSubcoreMesh(core_axis_name="c", subcore_axis_name="s")

    @functools.partial(
        pl.kernel,
        out_type=jax.ShapeDtypeStruct((m, 128), F32),
        mesh=mesh,
        scratch_types=[pltpu.VMEM((ch,), jnp.int32),
                       pltpu.VMEM((ch,), jnp.int32),
                       pltpu.VMEM((ch,), jnp.int32),
                       pltpu.VMEM((ch, 128), F32),
                       pltpu.SemaphoreType.DMA])
    def k(a_hbm, b_hbm, c_hbm, ia_hbm, ib_hbm, il_hbm, out, ia_v, ib_v,
          il_v, buf, sem):
        wid = lax.axis_index("s") * 2 + lax.axis_index("c")

        def step(ci, carry):
            g = ci * _NW + wid
            base = pl.multiple_of(g * ch, ch)
            pltpu.sync_copy(ia_hbm.at[pl.ds(base, ch)], ia_v)
            pltpu.sync_copy(ib_hbm.at[pl.ds(base, ch)], ib_v)
            pltpu.sync_copy(il_hbm.at[pl.ds(base, ch)], il_v)
            pltpu.async_copy(a_hbm.at[ia_v], buf, sem).wait()
            pltpu.async_copy(b_hbm.at[ib_v], buf, sem, add=True).wait()
            pltpu.async_copy(c_hbm.at[il_v], buf, sem, add=True).wait()
            pltpu.sync_copy(buf, out.at[pl.ds(base, ch)])
            return carry

        lax.fori_loop(0, per, step, 0)

    return k(tab_a, tab_b, lin, idx_a, idx_b, idx_l)


_SC_LINEAR = pltpu.CompilerParams(use_tc_tiling_on_sc=False)


def _sc_scatter_node(mg, idx):
    nch = N_EDGES // 128
    bound = math.ceil(nch / _NW)
    rows_t = N_NODES // 16
    mesh = plsc.VectorSubcoreMesh(core_axis_name="c", subcore_axis_name="s")

    @functools.partial(
        pl.kernel,
        out_type=jax.ShapeDtypeStruct((2, N_NODES, 64), F32),
        mesh=mesh,
        compiler_params=_SC_LINEAR,
        scratch_types=[pltpu.VMEM((rows_t, 64), F32),
                       pltpu.VMEM((128,), jnp.int32),
                       pltpu.VMEM((128, 64), F32),
                       pltpu.VMEM_SHARED((N_NODES, 64), F32),
                       pltpu.SemaphoreType.DMA])
    def k(mg_hbm, idx_hbm, out, zb, ibv, db, table, sem):
        cid = lax.axis_index("c")
        sid = lax.axis_index("s")
        wid = sid * 2 + cid

        def zrow(i, carry):
            for j in range(4):
                zb[i, pl.ds(j * 16, 16)] = jnp.zeros((16,), F32)
            return carry

        lax.fori_loop(0, rows_t, zrow, 0)
        pltpu.sync_copy(zb, table.at[pl.ds(sid * rows_t, rows_t)])
        plsc.subcore_barrier()

        def step(ci, carry):
            q = ci * _NW + wid

            @pl.when(q < nch)
            def _():
                base = pl.multiple_of(q * 128, 128)
                pltpu.sync_copy(idx_hbm.at[pl.ds(base, 128)], ibv)
                pltpu.sync_copy(mg_hbm.at[pl.ds(base, 128)], db)
                pltpu.sync_copy(db, table.at[ibv], add=True)

            return carry

        lax.fori_loop(0, bound, step, 0)
        plsc.subcore_barrier()
        pltpu.sync_copy(table.at[pl.ds(sid * rows_t, rows_t)],
                        out.at[cid, pl.ds(sid * rows_t, rows_t)])

    return k(mg, idx)


def _sc_scatter_edge(mga, mgb, idx):
    nch = N_LG // 128
    bound = math.ceil(nch / 16)
    rng = 20000
    rows_t = rng // 16
    mesh = plsc.VectorSubcoreMesh(core_axis_name="c", subcore_axis_name="s")

    @functools.partial(
        pl.kernel,
        out_type=[jax.ShapeDtypeStruct((N_EDGES, 32), F32),
                  jax.ShapeDtypeStruct((N_EDGES, 32), F32)],
        mesh=mesh,
        compiler_params=_SC_LINEAR,
        scratch_types=[pltpu.VMEM((rows_t, 32), F32),
                       pltpu.VMEM((128,), jnp.int32),
                       pltpu.VMEM((128,), jnp.int32),
                       pltpu.VMEM((128, 32), F32),
                       pltpu.VMEM_SHARED((rng + 8, 32), F32),
                       pltpu.SemaphoreType.DMA])
    def k(mga_hbm, mgb_hbm, idx_hbm, outa, outb, zb, ibv, ibw, db, table,
          sem):
        cid = lax.axis_index("c")
        sid = lax.axis_index("s")

        def zrow(i, carry):
            for j in range(2):
                zb[i, pl.ds(j * 16, 16)] = jnp.zeros((16,), F32)
            return carry

        lax.fori_loop(0, rows_t, zrow, 0)

        def one_half(mg_hbm, out):
            for r in range(8):
                lo = r * rng
                pltpu.sync_copy(zb, table.at[pl.ds(sid * rows_t, rows_t)])
                plsc.subcore_barrier()

                def step(ci, carry):
                    q = ci * 16 + sid

                    @pl.when(q < nch)
                    def _():
                        base = pl.multiple_of(q * 128, 128)
                        pltpu.sync_copy(idx_hbm.at[pl.ds(base, 128)], ibv)
                        for kk in range(8):
                            v = ibv[pl.ds(kk * 16, 16)]
                            rel = v - lo
                            ok = (rel >= 0) & (rel < rng)
                            ibw[pl.ds(kk * 16, 16)] = jnp.where(
                                ok, rel, jnp.int32(rng))
                        pltpu.sync_copy(mg_hbm.at[pl.ds(base, 128)], db)
                        pltpu.sync_copy(db, table.at[ibw], add=True)

                    return carry

                lax.fori_loop(0, bound, step, 0)
                plsc.subcore_barrier()
                pltpu.sync_copy(table.at[pl.ds(sid * rows_t, rows_t)],
                                out.at[pl.ds(lo + sid * rows_t, rows_t)])
                plsc.subcore_barrier()

        @pl.when(cid == 0)
        def _():
            one_half(mga_hbm, outa)

        @pl.when(cid == 1)
        def _():
            one_half(mgb_hbm, outb)

    return k(mga, mgb, idx)


def _rbf_fn(vmin, vmax, bins, ls, with_norm):
    step = (vmax - vmin) / (bins - 1)

    def f(blk):
        centers = vmin + lax.iota(jnp.int32, bins).reshape(
            1, bins).astype(F32) * step
        if with_norm:
            d = jnp.sqrt(jnp.sum(blk * blk, axis=1, keepdims=True))
        else:
            d = blk
        return jnp.exp(-(((d - centers) / ls) ** 2))

    return f


def _emb_chain(raw, pre_fn, p, m, bm):
    t1, st1 = _mm([(raw, pre_fn, None, None, p['W1'])], p['b1'], m, bm,
                  stats=True)
    a1, c1 = _aff_from_stats(st1, m, p['g1'], p['be1'])
    t2, st2 = _mm([(t1, None, (a1, c1), jax.nn.softplus, p['W2'])], p['b2'],
                  m, bm, stats=True)
    a2, c2 = _aff_from_stats(st2, m, p['g2'], p['be2'])
    return t2, (a2, c2)


def _cgcnn_node(lp, x, y, src, dst, arange, upd_stats=False):
    xs, xd = _mm_pair(x, None, None, lp['Ws'], lp['bs'], lp['Wd'], lp['bd'],
                      N_NODES, 1000)
    ey = _mm([(y, None, None, None, lp['We'])], lp['be'], N_EDGES, 1280)
    mm_arr = _sc_gather3(xs, xd, ey, src, dst, arange, N_EDGES)
    stm = _stats_only(mm_arr, N_EDGES, 3200)
    am, cm = _aff_from_stats(stm, N_EDGES, lp['gm'], lp['bm'])
    mg = _gate(mm_arr, am, cm, N_EDGES, 1280)
    aggp = _sc_scatter_node(mg, dst)
    agg, sta = _sum2_stats(aggp[0], aggp[1], N_NODES, 1000)
    an, cn = _aff_from_stats(sta, N_NODES, lp['gn'], lp['bn'])
    x2 = _res_update(agg, x, an, cn, 'sp_out', N_NODES, 1000,
                     stats=upd_stats)
    return x2, mg


def _cgcnn_edge(lp, y, ez, lsrc, ldst, arange):
    ys, yd = _mm_pair(y, None, None, lp['Ws'], lp['bs'], lp['Wd'], lp['bd'],
                      N_EDGES, 1280)
    mm_arr = _sc_gather3(ys, yd, ez, lsrc, ldst, arange, N_LG)
    stm = _stats_only(mm_arr, N_LG, 3200)
    am, cm = _aff_from_stats(stm, N_LG, lp['gm'], lp['bm'])
    mga, mgb = _gate(mm_arr, am, cm, N_LG, 1280, split=True)
    agga, aggb = _sc_scatter_edge(mga, mgb, ldst)
    sta = _stats_only2(agga, aggb, N_EDGES, 3200)
    an, cn = _aff_from_stats(sta, N_EDGES, lp['gn'], lp['bn'])
    return _res_update2(agga, aggb, y, an, cn, N_EDGES, 1280)


def kernel(atom_features, r, h, params, edge_src, edge_dst, lg_src, lg_dst):
    p = params
    src = edge_src.astype(jnp.int32)
    dst = edge_dst.astype(jnp.int32)
    lsrc = lg_src.astype(jnp.int32)
    ldst = lg_dst.astype(jnp.int32)
    arange = jnp.arange(N_LG, dtype=jnp.int32)

    t0, st0 = _mm([(atom_features, None, None, None, p['W_atom'])],
                  p['b_atom'], N_NODES, 1000, stats=True)
    a0, c0 = _aff_from_stats(st0, N_NODES, p['g_bn'], p['b_bn'])
    x = _ew_affine_act(t0, a0, c0, jax.nn.relu, N_NODES, 1000)

    t2e, affe = _emb_chain(r, _rbf_fn(0.0, 8.0, 40, 0.5, True),
                           p['edge_emb'], N_EDGES, 1280)
    y = _ew_affine_act(t2e, affe[0], affe[1], jax.nn.softplus, N_EDGES, 1280)

    t2a, affa = _emb_chain(h.reshape(-1, 1), _rbf_fn(-1.0, 1.0, 40, 0.1,
                                                     False),
                           p['angle_emb'], N_LG, 1280)
    ez = _mm_pair(t2a, affa, jax.nn.softplus,
                  p['layers'][0]['edge']['We'], p['layers'][0]['edge']['be'],
                  p['layers'][1]['edge']['We'], p['layers'][1]['edge']['be'],
                  N_LG, 1280)

    for li, lp in enumerate(p['layers']):
        x, mg = _cgcnn_node(lp['node'], x, y, src, dst, arange)
        wb_top = lp['Wb'][:64]
        wb_bot = lp['Wb'][64:]
        mbt, stb = _mm([(y, None, None, None, wb_top),
                        (mg, None, None, None, wb_bot)], lp['bb'],
                       N_EDGES, 1280, stats=True)
        ab, cb = _aff_from_stats(stb, N_EDGES, lp['gb'], lp['bbn'])
        y = _res_update(mbt, y, ab, cb, 'sp_in', N_EDGES, 1280)
        y = _cgcnn_edge(lp['edge'], y, ez[li], lsrc, ldst, arange)

    (xf, stf), _ = _cgcnn_node(p['final'], x, y, src, dst, arange,
                               upd_stats=True)
    af, cf = _aff_from_stats(stf, N_NODES, p['g_f'], p['b_f'])
    out = _pool_head(xf, af, cf, p['W_fc'], p['b_fc'], N_NODES, 1000)
    return out.reshape(())


# speedup vs baseline: 1.0009x; 1.0009x over previous
import functools
import math

import jax
import jax.numpy as jnp
import numpy as np
from jax import lax
from jax.experimental import pallas as pl
from jax.experimental.pallas import tpu as pltpu
from jax.experimental.pallas import tpu_sc as plsc

F32 = jnp.float32
EPS = 1e-5
N_NODES = 10000
N_EDGES = 160000
N_LG = 320000


def _row_spec(bm, k):
    return pl.BlockSpec((bm, k), lambda i: (i, 0))


def _const_spec(shape):
    nd = len(shape)
    return pl.BlockSpec(shape, lambda i: (0,) * nd)


def _aff_from_stats(st, m, g, b):
    s = st[0]
    q = st[1]
    mean = s / m
    var = q / m - mean * mean
    a = g * lax.rsqrt(var + EPS)
    c = b - mean * a
    return a, c


def _stats_update(st_ref, acc, n):
    @pl.when(pl.program_id(0) == 0)
    def _():
        st_ref[...] = jnp.zeros_like(st_ref)

    upd = jnp.concatenate(
        [jnp.sum(acc, axis=0)[None], jnp.sum(acc * acc, axis=0)[None],
         jnp.zeros((6, n), F32)], axis=0)
    st_ref[...] += upd


def _mm(inputs, bias, m, bm, stats=False):
    n_in = len(inputs)
    n_out = inputs[0][4].shape[1]
    grid = m // bm
    assert grid * bm == m, (m, bm)

    in_specs = []
    args = []
    has_aff = []
    pres = []
    acts = []
    for (x, pre, aff, act, w) in inputs:
        in_specs.append(_row_spec(bm, x.shape[1]))
        args.append(x)
        if aff is not None:
            kk = w.shape[0]
            in_specs += [_const_spec((1, kk)), _const_spec((1, kk))]
            args += [aff[0].reshape(1, kk), aff[1].reshape(1, kk)]
        in_specs.append(_const_spec(w.shape))
        args.append(w)
        has_aff.append(aff is not None)
        pres.append(pre)
        acts.append(act)
    in_specs.append(_const_spec((1, n_out)))
    args.append(bias.reshape(1, n_out))

    def body(*refs):
        i = 0
        acc = None
        for t in range(n_in):
            xb = refs[i][...]
            i += 1
            if pres[t] is not None:
                xb = pres[t](xb)
            if has_aff[t]:
                a = refs[i][...]
                c = refs[i + 1][...]
                i += 2
                xb = xb * a + c
            if acts[t] is not None:
                xb = acts[t](xb)
            w = refs[i][...]
            i += 1
            y = jnp.dot(xb, w, preferred_element_type=F32)
            acc = y if acc is None else acc + y
        acc = acc + refs[i][...]
        i += 1
        refs[i][...] = acc
        i += 1
        if stats:
            _stats_update(refs[i], acc, n_out)

    out_shape = [jax.ShapeDtypeStruct((m, n_out), F32)]
    out_specs = [_row_spec(bm, n_out)]
    if stats:
        out_shape.append(jax.ShapeDtypeStruct((8, n_out), F32))
        out_specs.append(_const_spec((8, n_out)))
    res = pl.pallas_call(
        body, grid=(grid,), in_specs=in_specs, out_specs=out_specs,
        out_shape=out_shape)(*args)
    return res if stats else res[0]


def _mm_pair(x, aff, act, w1, b1, w2, b2, m, bm):
    grid = m // bm
    assert grid * bm == m
    k = x.shape[1]
    n1 = w1.shape[1]
    n2 = w2.shape[1]
    in_specs = [_row_spec(bm, k)]
    args = [x]
    if aff is not None:
        in_specs += [_const_spec((1, k)), _const_spec((1, k))]
        args += [aff[0].reshape(1, k), aff[1].reshape(1, k)]
    in_specs += [_const_spec(w1.shape), _const_spec((1, n1)),
                 _const_spec(w2.shape), _const_spec((1, n2))]
    args += [w1, b1.reshape(1, n1), w2, b2.reshape(1, n2)]

    def body(*refs):
        i = 0
        xb = refs[i][...]
        i += 1
        if aff is not None:
            a = refs[i][...]
            c = refs[i + 1][...]
            i += 2
            xb = xb * a + c
        if act is not None:
            xb = act(xb)
        w1r, b1r, w2r, b2r = refs[i], refs[i + 1], refs[i + 2], refs[i + 3]
        o1, o2 = refs[i + 4], refs[i + 5]
        o1[...] = jnp.dot(xb, w1r[...], preferred_element_type=F32) + b1r[...]
        o2[...] = jnp.dot(xb, w2r[...], preferred_element_type=F32) + b2r[...]

    return pl.pallas_call(
        body, grid=(grid,), in_specs=in_specs,
        out_specs=[_row_spec(bm, n1), _row_spec(bm, n2)],
        out_shape=[jax.ShapeDtypeStruct((m, n1), F32),
                   jax.ShapeDtypeStruct((m, n2), F32)])(*args)


def _gate(mm_arr, a, c, m, bm, split=False):
    grid = m // bm
    half = mm_arr.shape[1] // 2

    def body(m_ref, a_ref, c_ref, *outs):
        n = m_ref[...] * a_ref[...] + c_ref[...]
        hf = n[:, :half]
        hs = n[:, half:]
        g = jax.nn.sigmoid(hf) * jax.nn.softplus(hs)
        if split:
            outs[0][...] = g[:, :half // 2]
            outs[1][...] = g[:, half // 2:]
        else:
            outs[0][...] = g

    if split:
        out_specs = [_row_spec(bm, half // 2), _row_spec(bm, half // 2)]
        out_shape = [jax.ShapeDtypeStruct((m, half // 2), F32),
                     jax.ShapeDtypeStruct((m, half // 2), F32)]
    else:
        out_specs = [_row_spec(bm, half)]
        out_shape = [jax.ShapeDtypeStruct((m, half), F32)]
    res = pl.pallas_call(
        body, grid=(grid,),
        in_specs=[_row_spec(bm, 2 * half), _const_spec((1, 2 * half)),
                  _const_spec((1, 2 * half))],
        out_specs=out_specs, out_shape=out_shape)(
            mm_arr, a.reshape(1, -1), c.reshape(1, -1))
    return res if split else res[0]


def _ew_affine_act(x, a, c, act, m, bm):
    grid = m // bm
    n = x.shape[1]

    def body(x_ref, a_ref, c_ref, o_ref):
        o_ref[...] = act(x_ref[...] * a_ref[...] + c_ref[...])

    return pl.pallas_call(
        body, grid=(grid,),
        in_specs=[_row_spec(bm, n), _const_spec((1, n)), _const_spec((1, n))],
        out_specs=[_row_spec(bm, n)],
        out_shape=[jax.ShapeDtypeStruct((m, n), F32)])(
            x, a.reshape(1, n), c.reshape(1, n))[0]


def _res_update(x, res, a, c, mode, m, bm, stats=False):
    grid = m // bm
    n = x.shape[1]

    def body(x_ref, r_ref, a_ref, c_ref, o_ref, *maybe_st):
        t = x_ref[...] * a_ref[...] + c_ref[...]
        if mode == 'sp_out':
            o = jax.nn.softplus(r_ref[...] + t)
        else:
            o = r_ref[...] + jax.nn.softplus(t)
        o_ref[...] = o
        if stats:
            _stats_update(maybe_st[0], o, n)

    out_shape = [jax.ShapeDtypeStruct((m, n), F32)]
    out_specs = [_row_spec(bm, n)]
    if stats:
        out_shape.append(jax.ShapeDtypeStruct((8, n), F32))
        out_specs.append(_const_spec((8, n)))
    res_ = pl.pallas_call(
        body, grid=(grid,),
        in_specs=[_row_spec(bm, n), _row_spec(bm, n), _const_spec((1, n)),
                  _const_spec((1, n))],
        out_specs=out_specs, out_shape=out_shape)(
            x, res, a.reshape(1, n), c.reshape(1, n))
    return res_ if stats else res_[0]


def _sum2_stats(a0, a1, m, bm):
    grid = m // bm
    n = a0.shape[1]

    def body(x_ref, y_ref, o_ref, st_ref):
        o = x_ref[...] + y_ref[...]
        o_ref[...] = o
        _stats_update(st_ref, o, n)

    return pl.pallas_call(
        body, grid=(grid,),
        in_specs=[_row_spec(bm, n), _row_spec(bm, n)],
        out_specs=[_row_spec(bm, n), _const_spec((8, n))],
        out_shape=[jax.ShapeDtypeStruct((m, n), F32),
                   jax.ShapeDtypeStruct((8, n), F32)])(a0, a1)


def _stats_only(x, m, bm):
    grid = m // bm
    n = x.shape[1]

    def body(x_ref, st_ref):
        _stats_update(st_ref, x_ref[...], n)

    return pl.pallas_call(
        body, grid=(grid,),
        in_specs=[_row_spec(bm, n)],
        out_specs=[_const_spec((8, n))],
        out_shape=[jax.ShapeDtypeStruct((8, n), F32)])(x)[0]


def _stats_only2(xa, xb, m, bm):
    grid = m // bm
    n = xa.shape[1] + xb.shape[1]

    def body(xa_ref, xb_ref, st_ref):
        x = jnp.concatenate([xa_ref[...], xb_ref[...]], axis=1)
        _stats_update(st_ref, x, n)

    return pl.pallas_call(
        body, grid=(grid,),
        in_specs=[_row_spec(bm, xa.shape[1]), _row_spec(bm, xb.shape[1])],
        out_specs=[_const_spec((8, n))],
        out_shape=[jax.ShapeDtypeStruct((8, n), F32)])(xa, xb)[0]


def _res_update2(xa, xb, res, a, c, m, bm):
    grid = m // bm
    n = xa.shape[1] + xb.shape[1]

    def body(xa_ref, xb_ref, r_ref, a_ref, c_ref, o_ref):
        x = jnp.concatenate([xa_ref[...], xb_ref[...]], axis=1)
        o_ref[...] = jax.nn.softplus(r_ref[...] + x * a_ref[...] + c_ref[...])

    return pl.pallas_call(
        body, grid=(grid,),
        in_specs=[_row_spec(bm, xa.shape[1]), _row_spec(bm, xb.shape[1]),
                  _row_spec(bm, n), _const_spec((1, n)), _const_spec((1, n))],
        out_specs=[_row_spec(bm, n)],
        out_shape=[jax.ShapeDtypeStruct((m, n), F32)])(
            xa, xb, res, a.reshape(1, n), c.reshape(1, n))[0]


def _pool_head(xf, a, c, wfc, bfc, m, bm):
    grid = m // bm
    n = xf.shape[1]

    def body(x_ref, a_ref, c_ref, w_ref, b_ref, o_ref):
        @pl.when(pl.program_id(0) == 0)
        def _():
            o_ref[...] = jnp.zeros_like(o_ref)

        hn = jax.nn.relu(x_ref[...] * a_ref[...] + c_ref[...])
        o_ref[...] += jnp.sum(jnp.dot(hn, w_ref[...],
                                      preferred_element_type=F32),
                              axis=0, keepdims=True)

        @pl.when(pl.program_id(0) == grid - 1)
        def _():
            o_ref[...] = o_ref[...] * (1.0 / m) + b_ref[...]

    return pl.pallas_call(
        body, grid=(grid,),
        in_specs=[_row_spec(bm, n), _const_spec((1, n)), _const_spec((1, n)),
                  _const_spec(wfc.shape), _const_spec((1, 1))],
        out_specs=[_const_spec((1, 1))],
        out_shape=[jax.ShapeDtypeStruct((1, 1), F32)])(
            xf, a.reshape(1, n), c.reshape(1, n), wfc, bfc.reshape(1, 1))[0]


_NW = 32


def _sc_gather3(tab_a, tab_b, lin, idx_a, idx_b, idx_l, m):
    ch = 80 if m % (80 * _NW) == 0 else 40
    nch = m // ch
    per = nch // _NW
    assert per * _NW == nch
    mesh = plsc.VectorSubcoreMesh(core_axis_name="c", subcore_axis_name="s")

    @functools.partial(
        pl.kernel,
        out_type=jax.ShapeDtypeStruct((m, 128), F32),
        mesh=mesh,
        scratch_types=[pltpu.VMEM((ch,), jnp.int32),
                       pltpu.VMEM((ch,), jnp.int32),
                       pltpu.VMEM((ch,), jnp.int32),
                       pltpu.VMEM((ch, 128), F32),
                       pltpu.SemaphoreType.DMA])
    def k(a_hbm, b_hbm, c_hbm, ia_hbm, ib_hbm, il_hbm, out, ia_v, ib_v,
          il_v, buf, sem):
        wid = lax.axis_index("s") * 2 + lax.axis_index("c")

        def step(ci, carry):
            g = ci * _NW + wid
            base = pl.multiple_of(g * ch, ch)
            pltpu.sync_copy(ia_hbm.at[pl.ds(base, ch)], ia_v)
            pltpu.sync_copy(ib_hbm.at[pl.ds(base, ch)], ib_v)
            pltpu.sync_copy(il_hbm.at[pl.ds(base, ch)], il_v)
            pltpu.async_copy(a_hbm.at[ia_v], buf, sem).wait()
            pltpu.async_copy(b_hbm.at[ib_v], buf, sem, add=True).wait()
            pltpu.async_copy(c_hbm.at[il_v], buf, sem, add=True).wait()
            pltpu.sync_copy(buf, out.at[pl.ds(base, ch)])
            return carry

        lax.fori_loop(0, per, step, 0)

    return k(tab_a, tab_b, lin, idx_a, idx_b, idx_l)


_SC_LINEAR = pltpu.CompilerParams(use_tc_tiling_on_sc=False)


def _sc_scatter_node(mg, idx):
    rows_t = N_NODES // 16
    mesh = plsc.VectorSubcoreMesh(core_axis_name="c", subcore_axis_name="s")
    grp = 4
    ngrp = 313
    full = 312
    tail = 2
    bound = math.ceil(ngrp / _NW)

    @functools.partial(
        pl.kernel,
        out_type=jax.ShapeDtypeStruct((2, N_NODES, 64), F32),
        mesh=mesh,
        compiler_params=_SC_LINEAR,
        scratch_types=[pltpu.VMEM((rows_t, 64), F32),
                       pltpu.VMEM((4, 128), jnp.int32),
                       pltpu.VMEM((512, 64), F32),
                       pltpu.VMEM_SHARED((N_NODES, 64), F32),
                       pltpu.SemaphoreType.DMA])
    def k(mg_hbm, idx_hbm, out, zb, ibv, db, table, sem):
        cid = lax.axis_index("c")
        sid = lax.axis_index("s")
        wid = sid * 2 + cid

        def zrow(i, carry):
            for j in range(4):
                zb[i, pl.ds(j * 16, 16)] = jnp.zeros((16,), F32)
            return carry

        lax.fori_loop(0, rows_t, zrow, 0)
        pltpu.sync_copy(zb, table.at[pl.ds(sid * rows_t, rows_t)])
        plsc.subcore_barrier()

        def group(q, n):
            goff = q * grp
            base = q * (grp * 128)
            pltpu.sync_copy(idx_hbm.at[pl.ds(goff, n)],
                            ibv.at[pl.ds(0, n)])
            pltpu.sync_copy(mg_hbm.at[pl.ds(base, n * 128)],
                            db.at[pl.ds(0, n * 128)])
            descs = [pltpu.async_copy(db.at[pl.ds(j * 128, 128)],
                                      table.at[ibv.at[j]], sem, add=True)
                     for j in range(n)]
            for d in descs:
                d.wait()

        def step(ci, carry):
            q = ci * _NW + wid

            @pl.when(q < full)
            def _():
                group(q, grp)

            @pl.when(q == full)
            def _():
                group(q, tail)

            return carry

        lax.fori_loop(0, bound, step, 0)
        plsc.subcore_barrier()
        pltpu.sync_copy(table.at[pl.ds(sid * rows_t, rows_t)],
                        out.at[cid, pl.ds(sid * rows_t, rows_t)])

    return k(mg, idx)


def _sc_scatter_edge(mga, mgb, idx):
    rng = 20000
    rows_t = rng // 16
    ngrp = 313
    full = 312
    tail = 4
    bound = math.ceil(ngrp / 16)
    mesh = plsc.VectorSubcoreMesh(core_axis_name="c", subcore_axis_name="s")

    @functools.partial(
        pl.kernel,
        out_type=[jax.ShapeDtypeStruct((N_EDGES, 32), F32),
                  jax.ShapeDtypeStruct((N_EDGES, 32), F32)],
        mesh=mesh,
        compiler_params=_SC_LINEAR,
        scratch_types=[pltpu.VMEM((rows_t, 32), F32),
                       pltpu.VMEM((8, 128), jnp.int32),
                       pltpu.VMEM((8, 128), jnp.int32),
                       pltpu.VMEM((1024, 32), F32),
                       pltpu.VMEM_SHARED((rng + 8, 32), F32),
                       pltpu.SemaphoreType.DMA])
    def k(mga_hbm, mgb_hbm, idx_hbm, outa, outb, zb, ibv, ibw, db, table,
          sem):
        cid = lax.axis_index("c")
        sid = lax.axis_index("s")

        def zrow(i, carry):
            for j in range(2):
                zb[i, pl.ds(j * 16, 16)] = jnp.zeros((16,), F32)
            return carry

        lax.fori_loop(0, rows_t, zrow, 0)

        def one_half(mg_hbm, out):
            for r in range(8):
                lo = r * rng
                pltpu.sync_copy(zb, table.at[pl.ds(sid * rows_t, rows_t)])
                plsc.subcore_barrier()

                def group(q, n):
                    goff = q * 8
                    base = q * 1024
                    pltpu.sync_copy(idx_hbm.at[pl.ds(goff, n)],
                                    ibv.at[pl.ds(0, n)])
                    pltpu.sync_copy(mg_hbm.at[pl.ds(base, n * 128)],
                                    db.at[pl.ds(0, n * 128)])
                    for j in range(n):
                        for kk in range(8):
                            v = ibv[j, pl.ds(kk * 16, 16)]
                            rel = v - lo
                            ok = (rel >= 0) & (rel < rng)
                            ibw[j, pl.ds(kk * 16, 16)] = jnp.where(
                                ok, rel, jnp.int32(rng))
                    descs = [pltpu.async_copy(db.at[pl.ds(j * 128, 128)],
                                              table.at[ibw.at[j]], sem,
                                              add=True)
                             for j in range(n)]
                    for d in descs:
                        d.wait()

                def step(ci, carry):
                    q = ci * 16 + sid

                    @pl.when(q < full)
                    def _():
                        group(q, 8)

                    @pl.when(q == full)
                    def _():
                        group(q, tail)

                    return carry

                lax.fori_loop(0, bound, step, 0)
                plsc.subcore_barrier()
                pltpu.sync_copy(table.at[pl.ds(sid * rows_t, rows_t)],
                                out.at[pl.ds(lo + sid * rows_t, rows_t)])
                plsc.subcore_barrier()

        @pl.when(cid == 0)
        def _():
            one_half(mga_hbm, outa)

        @pl.when(cid == 1)
        def _():
            one_half(mgb_hbm, outb)

    return k(mga, mgb, idx)


def _rbf_fn(vmin, vmax, bins, ls, with_norm):
    step = (vmax - vmin) / (bins - 1)

    def f(blk):
        centers = vmin + lax.iota(jnp.int32, bins).reshape(
            1, bins).astype(F32) * step
        if with_norm:
            d = jnp.sqrt(jnp.sum(blk * blk, axis=1, keepdims=True))
        else:
            d = blk
        return jnp.exp(-(((d - centers) / ls) ** 2))

    return f


def _emb_chain(raw, pre_fn, p, m, bm):
    t1, st1 = _mm([(raw, pre_fn, None, None, p['W1'])], p['b1'], m, bm,
                  stats=True)
    a1, c1 = _aff_from_stats(st1, m, p['g1'], p['be1'])
    t2, st2 = _mm([(t1, None, (a1, c1), jax.nn.softplus, p['W2'])], p['b2'],
                  m, bm, stats=True)
    a2, c2 = _aff_from_stats(st2, m, p['g2'], p['be2'])
    return t2, (a2, c2)


def _cgcnn_node(lp, x, y, src, dst, dst2d, arange, upd_stats=False):
    xs, xd = _mm_pair(x, None, None, lp['Ws'], lp['bs'], lp['Wd'], lp['bd'],
                      N_NODES, 1000)
    ey = _mm([(y, None, None, None, lp['We'])], lp['be'], N_EDGES, 1280)
    mm_arr = _sc_gather3(xs, xd, ey, src, dst, arange, N_EDGES)
    stm = _stats_only(mm_arr, N_EDGES, 3200)
    am, cm = _aff_from_stats(stm, N_EDGES, lp['gm'], lp['bm'])
    mg = _gate(mm_arr, am, cm, N_EDGES, 1280)
    aggp = _sc_scatter_node(mg, dst2d)
    agg, sta = _sum2_stats(aggp[0], aggp[1], N_NODES, 1000)
    an, cn = _aff_from_stats(sta, N_NODES, lp['gn'], lp['bn'])
    x2 = _res_update(agg, x, an, cn, 'sp_out', N_NODES, 1000,
                     stats=upd_stats)
    return x2, mg


def _cgcnn_edge(lp, y, ez, lsrc, ldst, ldst2d, arange):
    ys, yd = _mm_pair(y, None, None, lp['Ws'], lp['bs'], lp['Wd'], lp['bd'],
                      N_EDGES, 1280)
    mm_arr = _sc_gather3(ys, yd, ez, lsrc, ldst, arange, N_LG)
    stm = _stats_only(mm_arr, N_LG, 3200)
    am, cm = _aff_from_stats(stm, N_LG, lp['gm'], lp['bm'])
    mga, mgb = _gate(mm_arr, am, cm, N_LG, 1280, split=True)
    agga, aggb = _sc_scatter_edge(mga, mgb, ldst2d)
    sta = _stats_only2(agga, aggb, N_EDGES, 3200)
    an, cn = _aff_from_stats(sta, N_EDGES, lp['gn'], lp['bn'])
    return _res_update2(agga, aggb, y, an, cn, N_EDGES, 1280)


def kernel(atom_features, r, h, params, edge_src, edge_dst, lg_src, lg_dst):
    p = params
    src = edge_src.astype(jnp.int32)
    dst = edge_dst.astype(jnp.int32)
    lsrc = lg_src.astype(jnp.int32)
    ldst = lg_dst.astype(jnp.int32)
    arange = jnp.arange(N_LG, dtype=jnp.int32)
    dst2d = jnp.pad(dst.reshape(-1, 128), ((0, 6), (0, 0)))
    ldst2d = jnp.pad(ldst.reshape(-1, 128), ((0, 4), (0, 0)))

    t0, st0 = _mm([(atom_features, None, None, None, p['W_atom'])],
                  p['b_atom'], N_NODES, 1000, stats=True)
    a0, c0 = _aff_from_stats(st0, N_NODES, p['g_bn'], p['b_bn'])
    x = _ew_affine_act(t0, a0, c0, jax.nn.relu, N_NODES, 1000)

    t2e, affe = _emb_chain(r, _rbf_fn(0.0, 8.0, 40, 0.5, True),
                           p['edge_emb'], N_EDGES, 1280)
    y = _ew_affine_act(t2e, affe[0], affe[1], jax.nn.softplus, N_EDGES, 1280)

    t2a, affa = _emb_chain(h.reshape(-1, 1), _rbf_fn(-1.0, 1.0, 40, 0.1,
                                                     False),
                           p['angle_emb'], N_LG, 1280)
    ez = _mm_pair(t2a, affa, jax.nn.softplus,
                  p['layers'][0]['edge']['We'], p['layers'][0]['edge']['be'],
                  p['layers'][1]['edge']['We'], p['layers'][1]['edge']['be'],
                  N_LG, 1280)

    for li, lp in enumerate(p['layers']):
        x, mg = _cgcnn_node(lp['node'], x, y, src, dst, dst2d, arange)
        wb_top = lp['Wb'][:64]
        wb_bot = lp['Wb'][64:]
        mbt, stb = _mm([(y, None, None, None, wb_top),
                        (mg, None, None, None, wb_bot)], lp['bb'],
                       N_EDGES, 1280, stats=True)
        ab, cb = _aff_from_stats(stb, N_EDGES, lp['gb'], lp['bbn'])
        y = _res_update(mbt, y, ab, cb, 'sp_in', N_EDGES, 1280)
        y = _cgcnn_edge(lp['edge'], y, ez[li], lsrc, ldst, ldst2d, arange)

    (xf, stf), _ = _cgcnn_node(p['final'], x, y, src, dst, dst2d, arange,
                               upd_stats=True)
    af, cf = _aff_from_stats(stf, N_NODES, p['g_f'], p['b_f'])
    out = _pool_head(xf, af, cf, p['W_fc'], p['b_fc'], N_NODES, 1000)
    return out.reshape(())


# speedup vs baseline: 1.1310x; 1.1300x over previous
import functools
import math

import jax
import jax.numpy as jnp
import numpy as np
from jax import lax
from jax.experimental import pallas as pl
from jax.experimental.pallas import tpu as pltpu
from jax.experimental.pallas import tpu_sc as plsc

F32 = jnp.float32
EPS = 1e-5
N_NODES = 10000
N_EDGES = 160000
N_LG = 320000


def _row_spec(bm, k):
    return pl.BlockSpec((bm, k), lambda i: (i, 0))


def _const_spec(shape):
    nd = len(shape)
    return pl.BlockSpec(shape, lambda i: (0,) * nd)


def _aff_from_stats(st, m, g, b):
    s = st[0]
    q = st[1]
    mean = s / m
    var = q / m - mean * mean
    a = g * lax.rsqrt(var + EPS)
    c = b - mean * a
    return a, c


def _stats_update(st_ref, acc, n):
    @pl.when(pl.program_id(0) == 0)
    def _():
        st_ref[...] = jnp.zeros_like(st_ref)

    upd = jnp.concatenate(
        [jnp.sum(acc, axis=0)[None], jnp.sum(acc * acc, axis=0)[None],
         jnp.zeros((6, n), F32)], axis=0)
    st_ref[...] += upd


def _mm(inputs, bias, m, bm, stats=False):
    n_in = len(inputs)
    n_out = inputs[0][4].shape[1]
    grid = m // bm
    assert grid * bm == m, (m, bm)

    in_specs = []
    args = []
    has_aff = []
    pres = []
    acts = []
    for (x, pre, aff, act, w) in inputs:
        in_specs.append(_row_spec(bm, x.shape[1]))
        args.append(x)
        if aff is not None:
            kk = w.shape[0]
            in_specs += [_const_spec((1, kk)), _const_spec((1, kk))]
            args += [aff[0].reshape(1, kk), aff[1].reshape(1, kk)]
        in_specs.append(_const_spec(w.shape))
        args.append(w)
        has_aff.append(aff is not None)
        pres.append(pre)
        acts.append(act)
    in_specs.append(_const_spec((1, n_out)))
    args.append(bias.reshape(1, n_out))

    def body(*refs):
        i = 0
        acc = None
        for t in range(n_in):
            xb = refs[i][...]
            i += 1
            if pres[t] is not None:
                xb = pres[t](xb)
            if has_aff[t]:
                a = refs[i][...]
                c = refs[i + 1][...]
                i += 2
                xb = xb * a + c
            if acts[t] is not None:
                xb = acts[t](xb)
            w = refs[i][...]
            i += 1
            y = jnp.dot(xb, w, preferred_element_type=F32)
            acc = y if acc is None else acc + y
        acc = acc + refs[i][...]
        i += 1
        refs[i][...] = acc
        i += 1
        if stats:
            _stats_update(refs[i], acc, n_out)

    out_shape = [jax.ShapeDtypeStruct((m, n_out), F32)]
    out_specs = [_row_spec(bm, n_out)]
    if stats:
        out_shape.append(jax.ShapeDtypeStruct((8, n_out), F32))
        out_specs.append(_const_spec((8, n_out)))
    res = pl.pallas_call(
        body, grid=(grid,), in_specs=in_specs, out_specs=out_specs,
        out_shape=out_shape)(*args)
    return res if stats else res[0]


def _mm_pair(x, aff, act, w1, b1, w2, b2, m, bm):
    grid = m // bm
    assert grid * bm == m
    k = x.shape[1]
    n1 = w1.shape[1]
    n2 = w2.shape[1]
    in_specs = [_row_spec(bm, k)]
    args = [x]
    if aff is not None:
        in_specs += [_const_spec((1, k)), _const_spec((1, k))]
        args += [aff[0].reshape(1, k), aff[1].reshape(1, k)]
    in_specs += [_const_spec(w1.shape), _const_spec((1, n1)),
                 _const_spec(w2.shape), _const_spec((1, n2))]
    args += [w1, b1.reshape(1, n1), w2, b2.reshape(1, n2)]

    def body(*refs):
        i = 0
        xb = refs[i][...]
        i += 1
        if aff is not None:
            a = refs[i][...]
            c = refs[i + 1][...]
            i += 2
            xb = xb * a + c
        if act is not None:
            xb = act(xb)
        w1r, b1r, w2r, b2r = refs[i], refs[i + 1], refs[i + 2], refs[i + 3]
        o1, o2 = refs[i + 4], refs[i + 5]
        o1[...] = jnp.dot(xb, w1r[...], preferred_element_type=F32) + b1r[...]
        o2[...] = jnp.dot(xb, w2r[...], preferred_element_type=F32) + b2r[...]

    return pl.pallas_call(
        body, grid=(grid,), in_specs=in_specs,
        out_specs=[_row_spec(bm, n1), _row_spec(bm, n2)],
        out_shape=[jax.ShapeDtypeStruct((m, n1), F32),
                   jax.ShapeDtypeStruct((m, n2), F32)])(*args)


def _gate(mm_arr, a, c, m, bm, split=False):
    grid = m // bm
    half = mm_arr.shape[1] // 2

    def body(m_ref, a_ref, c_ref, *outs):
        n = m_ref[...] * a_ref[...] + c_ref[...]
        hf = n[:, :half]
        hs = n[:, half:]
        g = jax.nn.sigmoid(hf) * jax.nn.softplus(hs)
        if split:
            outs[0][...] = g[:, :half // 2]
            outs[1][...] = g[:, half // 2:]
        else:
            outs[0][...] = g

    if split:
        out_specs = [_row_spec(bm, half // 2), _row_spec(bm, half // 2)]
        out_shape = [jax.ShapeDtypeStruct((m, half // 2), F32),
                     jax.ShapeDtypeStruct((m, half // 2), F32)]
    else:
        out_specs = [_row_spec(bm, half)]
        out_shape = [jax.ShapeDtypeStruct((m, half), F32)]
    res = pl.pallas_call(
        body, grid=(grid,),
        in_specs=[_row_spec(bm, 2 * half), _const_spec((1, 2 * half)),
                  _const_spec((1, 2 * half))],
        out_specs=out_specs, out_shape=out_shape)(
            mm_arr, a.reshape(1, -1), c.reshape(1, -1))
    return res if split else res[0]


def _ew_affine_act(x, a, c, act, m, bm):
    grid = m // bm
    n = x.shape[1]

    def body(x_ref, a_ref, c_ref, o_ref):
        o_ref[...] = act(x_ref[...] * a_ref[...] + c_ref[...])

    return pl.pallas_call(
        body, grid=(grid,),
        in_specs=[_row_spec(bm, n), _const_spec((1, n)), _const_spec((1, n))],
        out_specs=[_row_spec(bm, n)],
        out_shape=[jax.ShapeDtypeStruct((m, n), F32)])(
            x, a.reshape(1, n), c.reshape(1, n))[0]


def _res_update(x, res, a, c, mode, m, bm, stats=False):
    grid = m // bm
    n = x.shape[1]

    def body(x_ref, r_ref, a_ref, c_ref, o_ref, *maybe_st):
        t = x_ref[...] * a_ref[...] + c_ref[...]
        if mode == 'sp_out':
            o = jax.nn.softplus(r_ref[...] + t)
        else:
            o = r_ref[...] + jax.nn.softplus(t)
        o_ref[...] = o
        if stats:
            _stats_update(maybe_st[0], o, n)

    out_shape = [jax.ShapeDtypeStruct((m, n), F32)]
    out_specs = [_row_spec(bm, n)]
    if stats:
        out_shape.append(jax.ShapeDtypeStruct((8, n), F32))
        out_specs.append(_const_spec((8, n)))
    res_ = pl.pallas_call(
        body, grid=(grid,),
        in_specs=[_row_spec(bm, n), _row_spec(bm, n), _const_spec((1, n)),
                  _const_spec((1, n))],
        out_specs=out_specs, out_shape=out_shape)(
            x, res, a.reshape(1, n), c.reshape(1, n))
    return res_ if stats else res_[0]


def _sum2_stats(a0, a1, m, bm):
    grid = m // bm
    n = a0.shape[1]

    def body(x_ref, y_ref, o_ref, st_ref):
        o = x_ref[...] + y_ref[...]
        o_ref[...] = o
        _stats_update(st_ref, o, n)

    return pl.pallas_call(
        body, grid=(grid,),
        in_specs=[_row_spec(bm, n), _row_spec(bm, n)],
        out_specs=[_row_spec(bm, n), _const_spec((8, n))],
        out_shape=[jax.ShapeDtypeStruct((m, n), F32),
                   jax.ShapeDtypeStruct((8, n), F32)])(a0, a1)


def _stats_only(x, m, bm):
    grid = m // bm
    n = x.shape[1]

    def body(x_ref, st_ref):
        _stats_update(st_ref, x_ref[...], n)

    return pl.pallas_call(
        body, grid=(grid,),
        in_specs=[_row_spec(bm, n)],
        out_specs=[_const_spec((8, n))],
        out_shape=[jax.ShapeDtypeStruct((8, n), F32)])(x)[0]


def _stats_only2(xa, xb, m, bm):
    grid = m // bm
    n = xa.shape[1] + xb.shape[1]

    def body(xa_ref, xb_ref, st_ref):
        x = jnp.concatenate([xa_ref[...], xb_ref[...]], axis=1)
        _stats_update(st_ref, x, n)

    return pl.pallas_call(
        body, grid=(grid,),
        in_specs=[_row_spec(bm, xa.shape[1]), _row_spec(bm, xb.shape[1])],
        out_specs=[_const_spec((8, n))],
        out_shape=[jax.ShapeDtypeStruct((8, n), F32)])(xa, xb)[0]


def _res_update2(xa, xb, res, a, c, m, bm):
    grid = m // bm
    n = xa.shape[1] + xb.shape[1]

    def body(xa_ref, xb_ref, r_ref, a_ref, c_ref, o_ref):
        x = jnp.concatenate([xa_ref[...], xb_ref[...]], axis=1)
        o_ref[...] = jax.nn.softplus(r_ref[...] + x * a_ref[...] + c_ref[...])

    return pl.pallas_call(
        body, grid=(grid,),
        in_specs=[_row_spec(bm, xa.shape[1]), _row_spec(bm, xb.shape[1]),
                  _row_spec(bm, n), _const_spec((1, n)), _const_spec((1, n))],
        out_specs=[_row_spec(bm, n)],
        out_shape=[jax.ShapeDtypeStruct((m, n), F32)])(
            xa, xb, res, a.reshape(1, n), c.reshape(1, n))[0]


def _pool_head(xf, a, c, wfc, bfc, m, bm):
    grid = m // bm
    n = xf.shape[1]

    def body(x_ref, a_ref, c_ref, w_ref, b_ref, o_ref):
        @pl.when(pl.program_id(0) == 0)
        def _():
            o_ref[...] = jnp.zeros_like(o_ref)

        hn = jax.nn.relu(x_ref[...] * a_ref[...] + c_ref[...])
        o_ref[...] += jnp.sum(jnp.dot(hn, w_ref[...],
                                      preferred_element_type=F32),
                              axis=0, keepdims=True)

        @pl.when(pl.program_id(0) == grid - 1)
        def _():
            o_ref[...] = o_ref[...] * (1.0 / m) + b_ref[...]

    return pl.pallas_call(
        body, grid=(grid,),
        in_specs=[_row_spec(bm, n), _const_spec((1, n)), _const_spec((1, n)),
                  _const_spec(wfc.shape), _const_spec((1, 1))],
        out_specs=[_const_spec((1, 1))],
        out_shape=[jax.ShapeDtypeStruct((1, 1), F32)])(
            xf, a.reshape(1, n), c.reshape(1, n), wfc, bfc.reshape(1, 1))[0]


_NW = 32


def _sc_gather3(tab_a, tab_b, lin, idx_a, idx_b, idx_l, m):
    ch = 768
    nfull = m // ch
    tail = m - nfull * ch
    ngrp = nfull + 1
    bound = math.ceil(ngrp / _NW)
    mesh = plsc.VectorSubcoreMesh(core_axis_name="c", subcore_axis_name="s")

    @functools.partial(
        pl.kernel,
        out_type=jax.ShapeDtypeStruct((m, 128), F32),
        mesh=mesh,
        scratch_types=[pltpu.VMEM((ch,), jnp.int32),
                       pltpu.VMEM((ch,), jnp.int32),
                       pltpu.VMEM((ch,), jnp.int32),
                       pltpu.VMEM((ch, 128), F32),
                       pltpu.SemaphoreType.DMA])
    def k(a_hbm, b_hbm, c_hbm, ia_hbm, ib_hbm, il_hbm, out, ia_v, ib_v,
          il_v, buf, sem):
        wid = lax.axis_index("s") * 2 + lax.axis_index("c")

        def chunk(g, n):
            nsub = n // 128
            base = pl.multiple_of(g * ch, 128)
            pltpu.sync_copy(ia_hbm.at[pl.ds(base, n)], ia_v.at[pl.ds(0, n)])
            pltpu.sync_copy(ib_hbm.at[pl.ds(base, n)], ib_v.at[pl.ds(0, n)])
            pltpu.sync_copy(il_hbm.at[pl.ds(base, n)], il_v.at[pl.ds(0, n)])
            for tab, iv, add in ((a_hbm, ia_v, False), (b_hbm, ib_v, True),
                                 (c_hbm, il_v, True)):
                descs = [pltpu.async_copy(
                    tab.at[iv.at[pl.ds(j * 128, 128)]],
                    buf.at[pl.ds(j * 128, 128)], sem, add=add)
                    for j in range(nsub)]
                for d in descs:
                    d.wait()
            pltpu.sync_copy(buf.at[pl.ds(0, n)], out.at[pl.ds(base, n)])

        def step(ci, carry):
            g = ci * _NW + wid

            @pl.when(g < nfull)
            def _():
                chunk(g, ch)

            @pl.when(g == nfull)
            def _():
                chunk(g, tail)

            return carry

        lax.fori_loop(0, bound, step, 0)

    return k(tab_a, tab_b, lin, idx_a, idx_b, idx_l)


_SC_LINEAR = pltpu.CompilerParams(use_tc_tiling_on_sc=False)


def _sc_scatter_node(mg, idx):
    rows_t = N_NODES // 16
    mesh = plsc.VectorSubcoreMesh(core_axis_name="c", subcore_axis_name="s")
    grp = 4
    ngrp = 313
    full = 312
    tail = 2
    bound = math.ceil(ngrp / _NW)

    @functools.partial(
        pl.kernel,
        out_type=jax.ShapeDtypeStruct((2, N_NODES, 64), F32),
        mesh=mesh,
        compiler_params=_SC_LINEAR,
        scratch_types=[pltpu.VMEM((rows_t, 64), F32),
                       pltpu.VMEM((4, 128), jnp.int32),
                       pltpu.VMEM((512, 64), F32),
                       pltpu.VMEM_SHARED((N_NODES, 64), F32),
                       pltpu.SemaphoreType.DMA])
    def k(mg_hbm, idx_hbm, out, zb, ibv, db, table, sem):
        cid = lax.axis_index("c")
        sid = lax.axis_index("s")
        wid = sid * 2 + cid

        def zrow(i, carry):
            for j in range(4):
                zb[i, pl.ds(j * 16, 16)] = jnp.zeros((16,), F32)
            return carry

        lax.fori_loop(0, rows_t, zrow, 0)
        pltpu.sync_copy(zb, table.at[pl.ds(sid * rows_t, rows_t)])
        plsc.subcore_barrier()

        def group(q, n):
            goff = q * grp
            base = q * (grp * 128)
            pltpu.sync_copy(idx_hbm.at[pl.ds(goff, n)],
                            ibv.at[pl.ds(0, n)])
            pltpu.sync_copy(mg_hbm.at[pl.ds(base, n * 128)],
                            db.at[pl.ds(0, n * 128)])
            descs = [pltpu.async_copy(db.at[pl.ds(j * 128, 128)],
                                      table.at[ibv.at[j]], sem, add=True)
                     for j in range(n)]
            for d in descs:
                d.wait()

        def step(ci, carry):
            q = ci * _NW + wid

            @pl.when(q < full)
            def _():
                group(q, grp)

            @pl.when(q == full)
            def _():
                group(q, tail)

            return carry

        lax.fori_loop(0, bound, step, 0)
        plsc.subcore_barrier()
        pltpu.sync_copy(table.at[pl.ds(sid * rows_t, rows_t)],
                        out.at[cid, pl.ds(sid * rows_t, rows_t)])

    return k(mg, idx)


def _sc_scatter_edge(mga, mgb, idx):
    rngs = [20000] * 8
    los = [sum(rngs[:i]) for i in range(8)]
    tmax = 20008
    ngrp = 313
    full = 312
    tail = 4
    bound = math.ceil(ngrp / 16)
    mesh = plsc.VectorSubcoreMesh(core_axis_name="c", subcore_axis_name="s")

    @functools.partial(
        pl.kernel,
        out_type=[jax.ShapeDtypeStruct((N_EDGES, 32), F32),
                  jax.ShapeDtypeStruct((N_EDGES, 32), F32)],
        mesh=mesh,
        compiler_params=_SC_LINEAR,
        scratch_types=[pltpu.VMEM((20000 // 16, 32), F32),
                       pltpu.VMEM((8, 128), jnp.int32),
                       pltpu.VMEM((8, 128), jnp.int32),
                       pltpu.VMEM((1024, 32), F32),
                       pltpu.VMEM_SHARED((tmax, 32), F32),
                       pltpu.SemaphoreType.DMA])
    def k(mga_hbm, mgb_hbm, idx_hbm, outa, outb, zb, ibv, ibw, db, table,
          sem):
        cid = lax.axis_index("c")
        sid = lax.axis_index("s")

        def zrow(i, carry):
            for j in range(2):
                zb[i, pl.ds(j * 16, 16)] = jnp.zeros((16,), F32)
            return carry

        lax.fori_loop(0, 20000 // 16, zrow, 0)

        def one_half(mg_hbm, out):
            for r in range(8):
                lo = los[r]
                rng = rngs[r]
                rows_t = rng // 16
                pltpu.sync_copy(zb.at[pl.ds(0, rows_t)],
                                table.at[pl.ds(sid * rows_t, rows_t)])
                plsc.subcore_barrier()

                def group(q, n):
                    goff = q * 8
                    base = q * 1024
                    pltpu.sync_copy(idx_hbm.at[pl.ds(goff, n)],
                                    ibv.at[pl.ds(0, n)])
                    pltpu.sync_copy(mg_hbm.at[pl.ds(base, n * 128)],
                                    db.at[pl.ds(0, n * 128)])
                    for j in range(n):
                        for kk in range(8):
                            v = ibv[j, pl.ds(kk * 16, 16)]
                            rel = v - lo
                            ok = (rel >= 0) & (rel < rng)
                            ibw[j, pl.ds(kk * 16, 16)] = jnp.where(
                                ok, rel, jnp.int32(rng))
                    descs = [pltpu.async_copy(db.at[pl.ds(j * 128, 128)],
                                              table.at[ibw.at[j]], sem,
                                              add=True)
                             for j in range(n)]
                    for d in descs:
                        d.wait()

                def step(ci, carry):
                    q = ci * 16 + sid

                    @pl.when(q < full)
                    def _():
                        group(q, 8)

                    @pl.when(q == full)
                    def _():
                        group(q, tail)

                    return carry

                lax.fori_loop(0, bound, step, 0)
                plsc.subcore_barrier()
                pltpu.sync_copy(table.at[pl.ds(sid * rows_t, rows_t)],
                                out.at[pl.ds(lo + sid * rows_t, rows_t)])
                plsc.subcore_barrier()

        @pl.when(cid == 0)
        def _():
            one_half(mga_hbm, outa)

        @pl.when(cid == 1)
        def _():
            one_half(mgb_hbm, outb)

    return k(mga, mgb, idx)


def _rbf_fn(vmin, vmax, bins, ls, with_norm):
    step = (vmax - vmin) / (bins - 1)

    def f(blk):
        centers = vmin + lax.iota(jnp.int32, bins).reshape(
            1, bins).astype(F32) * step
        if with_norm:
            d = jnp.sqrt(jnp.sum(blk * blk, axis=1, keepdims=True))
        else:
            d = blk
        return jnp.exp(-(((d - centers) / ls) ** 2))

    return f


def _emb_chain(raw, pre_fn, p, m, bm):
    t1, st1 = _mm([(raw, pre_fn, None, None, p['W1'])], p['b1'], m, bm,
                  stats=True)
    a1, c1 = _aff_from_stats(st1, m, p['g1'], p['be1'])
    t2, st2 = _mm([(t1, None, (a1, c1), jax.nn.softplus, p['W2'])], p['b2'],
                  m, bm, stats=True)
    a2, c2 = _aff_from_stats(st2, m, p['g2'], p['be2'])
    return t2, (a2, c2)


def _cgcnn_node(lp, x, y, src, dst, dst2d, arange, upd_stats=False):
    xs, xd = _mm_pair(x, None, None, lp['Ws'], lp['bs'], lp['Wd'], lp['bd'],
                      N_NODES, 1000)
    ey = _mm([(y, None, None, None, lp['We'])], lp['be'], N_EDGES, 1280)
    mm_arr = _sc_gather3(xs, xd, ey, src, dst, arange, N_EDGES)
    stm = _stats_only(mm_arr, N_EDGES, 3200)
    am, cm = _aff_from_stats(stm, N_EDGES, lp['gm'], lp['bm'])
    mg = _gate(mm_arr, am, cm, N_EDGES, 1280)
    aggp = _sc_scatter_node(mg, dst2d)
    agg, sta = _sum2_stats(aggp[0], aggp[1], N_NODES, 1000)
    an, cn = _aff_from_stats(sta, N_NODES, lp['gn'], lp['bn'])
    x2 = _res_update(agg, x, an, cn, 'sp_out', N_NODES, 1000,
                     stats=upd_stats)
    return x2, mg


def _cgcnn_edge(lp, y, ez, lsrc, ldst, ldst2d, arange):
    ys, yd = _mm_pair(y, None, None, lp['Ws'], lp['bs'], lp['Wd'], lp['bd'],
                      N_EDGES, 1280)
    mm_arr = _sc_gather3(ys, yd, ez, lsrc, ldst, arange, N_LG)
    stm = _stats_only(mm_arr, N_LG, 3200)
    am, cm = _aff_from_stats(stm, N_LG, lp['gm'], lp['bm'])
    mga, mgb = _gate(mm_arr, am, cm, N_LG, 1280, split=True)
    agga, aggb = _sc_scatter_edge(mga, mgb, ldst2d)
    sta = _stats_only2(agga, aggb, N_EDGES, 3200)
    an, cn = _aff_from_stats(sta, N_EDGES, lp['gn'], lp['bn'])
    return _res_update2(agga, aggb, y, an, cn, N_EDGES, 1280)


def kernel(atom_features, r, h, params, edge_src, edge_dst, lg_src, lg_dst):
    p = params
    src = edge_src.astype(jnp.int32)
    dst = edge_dst.astype(jnp.int32)
    lsrc = lg_src.astype(jnp.int32)
    ldst = lg_dst.astype(jnp.int32)
    arange = jnp.arange(N_LG, dtype=jnp.int32)
    dst2d = jnp.pad(dst.reshape(-1, 128), ((0, 6), (0, 0)))
    ldst2d = jnp.pad(ldst.reshape(-1, 128), ((0, 4), (0, 0)))

    t0, st0 = _mm([(atom_features, None, None, None, p['W_atom'])],
                  p['b_atom'], N_NODES, 1000, stats=True)
    a0, c0 = _aff_from_stats(st0, N_NODES, p['g_bn'], p['b_bn'])
    x = _ew_affine_act(t0, a0, c0, jax.nn.relu, N_NODES, 1000)

    t2e, affe = _emb_chain(r, _rbf_fn(0.0, 8.0, 40, 0.5, True),
                           p['edge_emb'], N_EDGES, 1280)
    y = _ew_affine_act(t2e, affe[0], affe[1], jax.nn.softplus, N_EDGES, 1280)

    t2a, affa = _emb_chain(h.reshape(-1, 1), _rbf_fn(-1.0, 1.0, 40, 0.1,
                                                     False),
                           p['angle_emb'], N_LG, 1280)
    ez = _mm_pair(t2a, affa, jax.nn.softplus,
                  p['layers'][0]['edge']['We'], p['layers'][0]['edge']['be'],
                  p['layers'][1]['edge']['We'], p['layers'][1]['edge']['be'],
                  N_LG, 1280)

    for li, lp in enumerate(p['layers']):
        x, mg = _cgcnn_node(lp['node'], x, y, src, dst, dst2d, arange)
        wb_top = lp['Wb'][:64]
        wb_bot = lp['Wb'][64:]
        mbt, stb = _mm([(y, None, None, None, wb_top),
                        (mg, None, None, None, wb_bot)], lp['bb'],
                       N_EDGES, 1280, stats=True)
        ab, cb = _aff_from_stats(stb, N_EDGES, lp['gb'], lp['bbn'])
        y = _res_update(mbt, y, ab, cb, 'sp_in', N_EDGES, 1280)
        y = _cgcnn_edge(lp['edge'], y, ez[li], lsrc, ldst, ldst2d, arange)

    (xf, stf), _ = _cgcnn_node(p['final'], x, y, src, dst, dst2d, arange,
                               upd_stats=True)
    af, cf = _aff_from_stats(stf, N_NODES, p['g_f'], p['b_f'])
    out = _pool_head(xf, af, cf, p['W_fc'], p['b_fc'], N_NODES, 1000)
    return out.reshape(())


# speedup vs baseline: 1.1368x; 1.0051x over previous
import functools
import math

import jax
import jax.numpy as jnp
import numpy as np
from jax import lax
from jax.experimental import pallas as pl
from jax.experimental.pallas import tpu as pltpu
from jax.experimental.pallas import tpu_sc as plsc

F32 = jnp.float32
EPS = 1e-5
N_NODES = 10000
N_EDGES = 160000
N_LG = 320000


def _row_spec(bm, k):
    return pl.BlockSpec((bm, k), lambda i: (i, 0))


def _const_spec(shape):
    nd = len(shape)
    return pl.BlockSpec(shape, lambda i: (0,) * nd)


def _aff_from_stats(st, m, g, b):
    s = st[0]
    q = st[1]
    mean = s / m
    var = q / m - mean * mean
    a = g * lax.rsqrt(var + EPS)
    c = b - mean * a
    return a, c


def _stats_update(st_ref, acc, n):
    @pl.when(pl.program_id(0) == 0)
    def _():
        st_ref[...] = jnp.zeros_like(st_ref)

    upd = jnp.concatenate(
        [jnp.sum(acc, axis=0)[None], jnp.sum(acc * acc, axis=0)[None],
         jnp.zeros((6, n), F32)], axis=0)
    st_ref[...] += upd


def _mm(inputs, bias, m, bm, stats=False):
    n_in = len(inputs)
    n_out = inputs[0][4].shape[1]
    grid = m // bm
    assert grid * bm == m, (m, bm)

    in_specs = []
    args = []
    has_aff = []
    pres = []
    acts = []
    for (x, pre, aff, act, w) in inputs:
        in_specs.append(_row_spec(bm, x.shape[1]))
        args.append(x)
        if aff is not None:
            kk = w.shape[0]
            in_specs += [_const_spec((1, kk)), _const_spec((1, kk))]
            args += [aff[0].reshape(1, kk), aff[1].reshape(1, kk)]
        in_specs.append(_const_spec(w.shape))
        args.append(w)
        has_aff.append(aff is not None)
        pres.append(pre)
        acts.append(act)
    in_specs.append(_const_spec((1, n_out)))
    args.append(bias.reshape(1, n_out))

    def body(*refs):
        i = 0
        acc = None
        for t in range(n_in):
            xb = refs[i][...]
            i += 1
            if pres[t] is not None:
                xb = pres[t](xb)
            if has_aff[t]:
                a = refs[i][...]
                c = refs[i + 1][...]
                i += 2
                xb = xb * a + c
            if acts[t] is not None:
                xb = acts[t](xb)
            w = refs[i][...]
            i += 1
            y = jnp.dot(xb, w, preferred_element_type=F32)
            acc = y if acc is None else acc + y
        acc = acc + refs[i][...]
        i += 1
        refs[i][...] = acc
        i += 1
        if stats:
            _stats_update(refs[i], acc, n_out)

    out_shape = [jax.ShapeDtypeStruct((m, n_out), F32)]
    out_specs = [_row_spec(bm, n_out)]
    if stats:
        out_shape.append(jax.ShapeDtypeStruct((8, n_out), F32))
        out_specs.append(_const_spec((8, n_out)))
    res = pl.pallas_call(
        body, grid=(grid,), in_specs=in_specs, out_specs=out_specs,
        out_shape=out_shape)(*args)
    return res if stats else res[0]


def _mm_pair(x, aff, act, w1, b1, w2, b2, m, bm):
    grid = m // bm
    assert grid * bm == m
    k = x.shape[1]
    n1 = w1.shape[1]
    n2 = w2.shape[1]
    in_specs = [_row_spec(bm, k)]
    args = [x]
    if aff is not None:
        in_specs += [_const_spec((1, k)), _const_spec((1, k))]
        args += [aff[0].reshape(1, k), aff[1].reshape(1, k)]
    in_specs += [_const_spec(w1.shape), _const_spec((1, n1)),
                 _const_spec(w2.shape), _const_spec((1, n2))]
    args += [w1, b1.reshape(1, n1), w2, b2.reshape(1, n2)]

    def body(*refs):
        i = 0
        xb = refs[i][...]
        i += 1
        if aff is not None:
            a = refs[i][...]
            c = refs[i + 1][...]
            i += 2
            xb = xb * a + c
        if act is not None:
            xb = act(xb)
        w1r, b1r, w2r, b2r = refs[i], refs[i + 1], refs[i + 2], refs[i + 3]
        o1, o2 = refs[i + 4], refs[i + 5]
        o1[...] = jnp.dot(xb, w1r[...], preferred_element_type=F32) + b1r[...]
        o2[...] = jnp.dot(xb, w2r[...], preferred_element_type=F32) + b2r[...]

    return pl.pallas_call(
        body, grid=(grid,), in_specs=in_specs,
        out_specs=[_row_spec(bm, n1), _row_spec(bm, n2)],
        out_shape=[jax.ShapeDtypeStruct((m, n1), F32),
                   jax.ShapeDtypeStruct((m, n2), F32)])(*args)


def _gate(mm_arr, a, c, m, bm, split=False):
    grid = m // bm
    half = mm_arr.shape[1] // 2

    def body(m_ref, a_ref, c_ref, *outs):
        n = m_ref[...] * a_ref[...] + c_ref[...]
        hf = n[:, :half]
        hs = n[:, half:]
        g = jax.nn.sigmoid(hf) * jax.nn.softplus(hs)
        if split:
            outs[0][...] = g[:, :half // 2]
            outs[1][...] = g[:, half // 2:]
        else:
            outs[0][...] = g

    if split:
        out_specs = [_row_spec(bm, half // 2), _row_spec(bm, half // 2)]
        out_shape = [jax.ShapeDtypeStruct((m, half // 2), F32),
                     jax.ShapeDtypeStruct((m, half // 2), F32)]
    else:
        out_specs = [_row_spec(bm, half)]
        out_shape = [jax.ShapeDtypeStruct((m, half), F32)]
    res = pl.pallas_call(
        body, grid=(grid,),
        in_specs=[_row_spec(bm, 2 * half), _const_spec((1, 2 * half)),
                  _const_spec((1, 2 * half))],
        out_specs=out_specs, out_shape=out_shape)(
            mm_arr, a.reshape(1, -1), c.reshape(1, -1))
    return res if split else res[0]


def _ew_affine_act(x, a, c, act, m, bm):
    grid = m // bm
    n = x.shape[1]

    def body(x_ref, a_ref, c_ref, o_ref):
        o_ref[...] = act(x_ref[...] * a_ref[...] + c_ref[...])

    return pl.pallas_call(
        body, grid=(grid,),
        in_specs=[_row_spec(bm, n), _const_spec((1, n)), _const_spec((1, n))],
        out_specs=[_row_spec(bm, n)],
        out_shape=[jax.ShapeDtypeStruct((m, n), F32)])(
            x, a.reshape(1, n), c.reshape(1, n))[0]


def _res_update(x, res, a, c, mode, m, bm, stats=False):
    grid = m // bm
    n = x.shape[1]

    def body(x_ref, r_ref, a_ref, c_ref, o_ref, *maybe_st):
        t = x_ref[...] * a_ref[...] + c_ref[...]
        if mode == 'sp_out':
            o = jax.nn.softplus(r_ref[...] + t)
        else:
            o = r_ref[...] + jax.nn.softplus(t)
        o_ref[...] = o
        if stats:
            _stats_update(maybe_st[0], o, n)

    out_shape = [jax.ShapeDtypeStruct((m, n), F32)]
    out_specs = [_row_spec(bm, n)]
    if stats:
        out_shape.append(jax.ShapeDtypeStruct((8, n), F32))
        out_specs.append(_const_spec((8, n)))
    res_ = pl.pallas_call(
        body, grid=(grid,),
        in_specs=[_row_spec(bm, n), _row_spec(bm, n), _const_spec((1, n)),
                  _const_spec((1, n))],
        out_specs=out_specs, out_shape=out_shape)(
            x, res, a.reshape(1, n), c.reshape(1, n))
    return res_ if stats else res_[0]


def _sum2_stats(a0, a1, m, bm):
    grid = m // bm
    n = a0.shape[1]

    def body(x_ref, y_ref, o_ref, st_ref):
        o = x_ref[...] + y_ref[...]
        o_ref[...] = o
        _stats_update(st_ref, o, n)

    return pl.pallas_call(
        body, grid=(grid,),
        in_specs=[_row_spec(bm, n), _row_spec(bm, n)],
        out_specs=[_row_spec(bm, n), _const_spec((8, n))],
        out_shape=[jax.ShapeDtypeStruct((m, n), F32),
                   jax.ShapeDtypeStruct((8, n), F32)])(a0, a1)


def _stats_only(x, m, bm):
    grid = m // bm
    n = x.shape[1]

    def body(x_ref, st_ref):
        _stats_update(st_ref, x_ref[...], n)

    return pl.pallas_call(
        body, grid=(grid,),
        in_specs=[_row_spec(bm, n)],
        out_specs=[_const_spec((8, n))],
        out_shape=[jax.ShapeDtypeStruct((8, n), F32)])(x)[0]


def _stats_only2(xa, xb, m, bm):
    grid = m // bm
    n = xa.shape[1] + xb.shape[1]

    def body(xa_ref, xb_ref, st_ref):
        x = jnp.concatenate([xa_ref[...], xb_ref[...]], axis=1)
        _stats_update(st_ref, x, n)

    return pl.pallas_call(
        body, grid=(grid,),
        in_specs=[_row_spec(bm, xa.shape[1]), _row_spec(bm, xb.shape[1])],
        out_specs=[_const_spec((8, n))],
        out_shape=[jax.ShapeDtypeStruct((8, n), F32)])(xa, xb)[0]


def _res_update2(xa, xb, res, a, c, m, bm):
    grid = m // bm
    n = xa.shape[1] + xb.shape[1]

    def body(xa_ref, xb_ref, r_ref, a_ref, c_ref, o_ref):
        x = jnp.concatenate([xa_ref[...], xb_ref[...]], axis=1)
        o_ref[...] = jax.nn.softplus(r_ref[...] + x * a_ref[...] + c_ref[...])

    return pl.pallas_call(
        body, grid=(grid,),
        in_specs=[_row_spec(bm, xa.shape[1]), _row_spec(bm, xb.shape[1]),
                  _row_spec(bm, n), _const_spec((1, n)), _const_spec((1, n))],
        out_specs=[_row_spec(bm, n)],
        out_shape=[jax.ShapeDtypeStruct((m, n), F32)])(
            xa, xb, res, a.reshape(1, n), c.reshape(1, n))[0]


def _pool_head(xf, a, c, wfc, bfc, m, bm):
    grid = m // bm
    n = xf.shape[1]

    def body(x_ref, a_ref, c_ref, w_ref, b_ref, o_ref):
        @pl.when(pl.program_id(0) == 0)
        def _():
            o_ref[...] = jnp.zeros_like(o_ref)

        hn = jax.nn.relu(x_ref[...] * a_ref[...] + c_ref[...])
        o_ref[...] += jnp.sum(jnp.dot(hn, w_ref[...],
                                      preferred_element_type=F32),
                              axis=0, keepdims=True)

        @pl.when(pl.program_id(0) == grid - 1)
        def _():
            o_ref[...] = o_ref[...] * (1.0 / m) + b_ref[...]

    return pl.pallas_call(
        body, grid=(grid,),
        in_specs=[_row_spec(bm, n), _const_spec((1, n)), _const_spec((1, n)),
                  _const_spec(wfc.shape), _const_spec((1, 1))],
        out_specs=[_const_spec((1, 1))],
        out_shape=[jax.ShapeDtypeStruct((1, 1), F32)])(
            xf, a.reshape(1, n), c.reshape(1, n), wfc, bfc.reshape(1, 1))[0]


_NW = 32


def _sc_gather3(tab_a, tab_b, lin, idx_a, idx_b, idx_l, m):
    ch = 768
    nfull = m // ch
    tail = m - nfull * ch
    ngrp = nfull + 1
    bound = math.ceil(ngrp / _NW)
    mesh = plsc.VectorSubcoreMesh(core_axis_name="c", subcore_axis_name="s")

    @functools.partial(
        pl.kernel,
        out_type=jax.ShapeDtypeStruct((m, 128), F32),
        mesh=mesh,
        scratch_types=[pltpu.VMEM((ch,), jnp.int32),
                       pltpu.VMEM((ch,), jnp.int32),
                       pltpu.VMEM((ch,), jnp.int32),
                       pltpu.VMEM((ch, 128), F32),
                       pltpu.SemaphoreType.DMA])
    def k(a_hbm, b_hbm, c_hbm, ia_hbm, ib_hbm, il_hbm, out, ia_v, ib_v,
          il_v, buf, sem):
        wid = lax.axis_index("s") * 2 + lax.axis_index("c")

        def chunk(g, n):
            nsub = n // 128
            base = pl.multiple_of(g * ch, 128)
            idescs = [pltpu.async_copy(src.at[pl.ds(base, n)],
                                       dst.at[pl.ds(0, n)], sem)
                      for src, dst in ((ia_hbm, ia_v), (ib_hbm, ib_v),
                                       (il_hbm, il_v))]
            for d in idescs:
                d.wait()
            for tab, iv, add in ((a_hbm, ia_v, False), (b_hbm, ib_v, True),
                                 (c_hbm, il_v, True)):
                descs = [pltpu.async_copy(
                    tab.at[iv.at[pl.ds(j * 128, 128)]],
                    buf.at[pl.ds(j * 128, 128)], sem, add=add)
                    for j in range(nsub)]
                for d in descs:
                    d.wait()
            pltpu.sync_copy(buf.at[pl.ds(0, n)], out.at[pl.ds(base, n)])

        def step(ci, carry):
            g = ci * _NW + wid

            @pl.when(g < nfull)
            def _():
                chunk(g, ch)

            @pl.when(g == nfull)
            def _():
                chunk(g, tail)

            return carry

        lax.fori_loop(0, bound, step, 0)

    return k(tab_a, tab_b, lin, idx_a, idx_b, idx_l)


_SC_LINEAR = pltpu.CompilerParams(use_tc_tiling_on_sc=False)


def _sc_scatter_node(mg, idx):
    rows_t = N_NODES // 16
    mesh = plsc.VectorSubcoreMesh(core_axis_name="c", subcore_axis_name="s")
    grp = 4
    ngrp = 313
    full = 312
    tail = 2
    bound = math.ceil(ngrp / _NW)

    @functools.partial(
        pl.kernel,
        out_type=jax.ShapeDtypeStruct((2, N_NODES, 64), F32),
        mesh=mesh,
        compiler_params=_SC_LINEAR,
        scratch_types=[pltpu.VMEM((rows_t, 64), F32),
                       pltpu.VMEM((4, 128), jnp.int32),
                       pltpu.VMEM((512, 64), F32),
                       pltpu.VMEM_SHARED((N_NODES, 64), F32),
                       pltpu.SemaphoreType.DMA])
    def k(mg_hbm, idx_hbm, out, zb, ibv, db, table, sem):
        cid = lax.axis_index("c")
        sid = lax.axis_index("s")
        wid = sid * 2 + cid

        def zrow(i, carry):
            for j in range(4):
                zb[i, pl.ds(j * 16, 16)] = jnp.zeros((16,), F32)
            return carry

        lax.fori_loop(0, rows_t, zrow, 0)
        pltpu.sync_copy(zb, table.at[pl.ds(sid * rows_t, rows_t)])
        plsc.subcore_barrier()

        def group(q, n):
            goff = q * grp
            base = q * (grp * 128)
            ldescs = [pltpu.async_copy(idx_hbm.at[pl.ds(goff, n)],
                                       ibv.at[pl.ds(0, n)], sem),
                      pltpu.async_copy(mg_hbm.at[pl.ds(base, n * 128)],
                                       db.at[pl.ds(0, n * 128)], sem)]
            for d in ldescs:
                d.wait()
            descs = [pltpu.async_copy(db.at[pl.ds(j * 128, 128)],
                                      table.at[ibv.at[j]], sem, add=True)
                     for j in range(n)]
            for d in descs:
                d.wait()

        def step(ci, carry):
            q = ci * _NW + wid

            @pl.when(q < full)
            def _():
                group(q, grp)

            @pl.when(q == full)
            def _():
                group(q, tail)

            return carry

        lax.fori_loop(0, bound, step, 0)
        plsc.subcore_barrier()
        pltpu.sync_copy(table.at[pl.ds(sid * rows_t, rows_t)],
                        out.at[cid, pl.ds(sid * rows_t, rows_t)])

    return k(mg, idx)


def _sc_scatter_edge(mga, mgb, idx):
    rngs = [20000] * 8
    los = [sum(rngs[:i]) for i in range(8)]
    tmax = 20008
    ngrp = 313
    full = 312
    tail = 4
    bound = math.ceil(ngrp / 16)
    mesh = plsc.VectorSubcoreMesh(core_axis_name="c", subcore_axis_name="s")

    @functools.partial(
        pl.kernel,
        out_type=[jax.ShapeDtypeStruct((N_EDGES, 32), F32),
                  jax.ShapeDtypeStruct((N_EDGES, 32), F32)],
        mesh=mesh,
        compiler_params=_SC_LINEAR,
        scratch_types=[pltpu.VMEM((20000 // 16, 32), F32),
                       pltpu.VMEM((8, 128), jnp.int32),
                       pltpu.VMEM((8, 128), jnp.int32),
                       pltpu.VMEM((1024, 32), F32),
                       pltpu.VMEM_SHARED((tmax, 32), F32),
                       pltpu.SemaphoreType.DMA])
    def k(mga_hbm, mgb_hbm, idx_hbm, outa, outb, zb, ibv, ibw, db, table,
          sem):
        cid = lax.axis_index("c")
        sid = lax.axis_index("s")

        def zrow(i, carry):
            for j in range(2):
                zb[i, pl.ds(j * 16, 16)] = jnp.zeros((16,), F32)
            return carry

        lax.fori_loop(0, 20000 // 16, zrow, 0)

        def one_half(mg_hbm, out):
            for r in range(8):
                lo = los[r]
                rng = rngs[r]
                rows_t = rng // 16
                pltpu.sync_copy(zb.at[pl.ds(0, rows_t)],
                                table.at[pl.ds(sid * rows_t, rows_t)])
                plsc.subcore_barrier()

                def group(q, n):
                    goff = q * 8
                    base = q * 1024
                    ldescs = [pltpu.async_copy(idx_hbm.at[pl.ds(goff, n)],
                                               ibv.at[pl.ds(0, n)], sem),
                              pltpu.async_copy(
                                  mg_hbm.at[pl.ds(base, n * 128)],
                                  db.at[pl.ds(0, n * 128)], sem)]
                    for d in ldescs:
                        d.wait()
                    for j in range(n):
                        for kk in range(8):
                            v = ibv[j, pl.ds(kk * 16, 16)]
                            rel = v - lo
                            ok = (rel >= 0) & (rel < rng)
                            ibw[j, pl.ds(kk * 16, 16)] = jnp.where(
                                ok, rel, jnp.int32(rng))
                    descs = [pltpu.async_copy(db.at[pl.ds(j * 128, 128)],
                                              table.at[ibw.at[j]], sem,
                                              add=True)
                             for j in range(n)]
                    for d in descs:
                        d.wait()

                def step(ci, carry):
                    q = ci * 16 + sid

                    @pl.when(q < full)
                    def _():
                        group(q, 8)

                    @pl.when(q == full)
                    def _():
                        group(q, tail)

                    return carry

                lax.fori_loop(0, bound, step, 0)
                plsc.subcore_barrier()
                pltpu.sync_copy(table.at[pl.ds(sid * rows_t, rows_t)],
                                out.at[pl.ds(lo + sid * rows_t, rows_t)])
                plsc.subcore_barrier()

        @pl.when(cid == 0)
        def _():
            one_half(mga_hbm, outa)

        @pl.when(cid == 1)
        def _():
            one_half(mgb_hbm, outb)

    return k(mga, mgb, idx)


def _rbf_fn(vmin, vmax, bins, ls, with_norm):
    step = (vmax - vmin) / (bins - 1)

    def f(blk):
        centers = vmin + lax.iota(jnp.int32, bins).reshape(
            1, bins).astype(F32) * step
        if with_norm:
            d = jnp.sqrt(jnp.sum(blk * blk, axis=1, keepdims=True))
        else:
            d = blk
        return jnp.exp(-(((d - centers) / ls) ** 2))

    return f


def _emb_chain(raw, pre_fn, p, m, bm):
    t1, st1 = _mm([(raw, pre_fn, None, None, p['W1'])], p['b1'], m, bm,
                  stats=True)
    a1, c1 = _aff_from_stats(st1, m, p['g1'], p['be1'])
    t2, st2 = _mm([(t1, None, (a1, c1), jax.nn.softplus, p['W2'])], p['b2'],
                  m, bm, stats=True)
    a2, c2 = _aff_from_stats(st2, m, p['g2'], p['be2'])
    return t2, (a2, c2)


def _cgcnn_node(lp, x, y, src, dst, dst2d, arange, upd_stats=False):
    xs, xd = _mm_pair(x, None, None, lp['Ws'], lp['bs'], lp['Wd'], lp['bd'],
                      N_NODES, 1000)
    ey = _mm([(y, None, None, None, lp['We'])], lp['be'], N_EDGES, 1280)
    mm_arr = _sc_gather3(xs, xd, ey, src, dst, arange, N_EDGES)
    stm = _stats_only(mm_arr, N_EDGES, 3200)
    am, cm = _aff_from_stats(stm, N_EDGES, lp['gm'], lp['bm'])
    mg = _gate(mm_arr, am, cm, N_EDGES, 1280)
    aggp = _sc_scatter_node(mg, dst2d)
    agg, sta = _sum2_stats(aggp[0], aggp[1], N_NODES, 1000)
    an, cn = _aff_from_stats(sta, N_NODES, lp['gn'], lp['bn'])
    x2 = _res_update(agg, x, an, cn, 'sp_out', N_NODES, 1000,
                     stats=upd_stats)
    return x2, mg


def _cgcnn_edge(lp, y, ez, lsrc, ldst, ldst2d, arange):
    ys, yd = _mm_pair(y, None, None, lp['Ws'], lp['bs'], lp['Wd'], lp['bd'],
                      N_EDGES, 1280)
    mm_arr = _sc_gather3(ys, yd, ez, lsrc, ldst, arange, N_LG)
    stm = _stats_only(mm_arr, N_LG, 3200)
    am, cm = _aff_from_stats(stm, N_LG, lp['gm'], lp['bm'])
    mga, mgb = _gate(mm_arr, am, cm, N_LG, 1280, split=True)
    agga, aggb = _sc_scatter_edge(mga, mgb, ldst2d)
    sta = _stats_only2(agga, aggb, N_EDGES, 3200)
    an, cn = _aff_from_stats(sta, N_EDGES, lp['gn'], lp['bn'])
    return _res_update2(agga, aggb, y, an, cn, N_EDGES, 1280)


def kernel(atom_features, r, h, params, edge_src, edge_dst, lg_src, lg_dst):
    p = params
    src = edge_src.astype(jnp.int32)
    dst = edge_dst.astype(jnp.int32)
    lsrc = lg_src.astype(jnp.int32)
    ldst = lg_dst.astype(jnp.int32)
    arange = jnp.arange(N_LG, dtype=jnp.int32)
    dst2d = jnp.pad(dst.reshape(-1, 128), ((0, 6), (0, 0)))
    ldst2d = jnp.pad(ldst.reshape(-1, 128), ((0, 4), (0, 0)))

    t0, st0 = _mm([(atom_features, None, None, None, p['W_atom'])],
                  p['b_atom'], N_NODES, 1000, stats=True)
    a0, c0 = _aff_from_stats(st0, N_NODES, p['g_bn'], p['b_bn'])
    x = _ew_affine_act(t0, a0, c0, jax.nn.relu, N_NODES, 1000)

    t2e, affe = _emb_chain(r, _rbf_fn(0.0, 8.0, 40, 0.5, True),
                           p['edge_emb'], N_EDGES, 1280)
    y = _ew_affine_act(t2e, affe[0], affe[1], jax.nn.softplus, N_EDGES, 1280)

    t2a, affa = _emb_chain(h.reshape(-1, 1), _rbf_fn(-1.0, 1.0, 40, 0.1,
                                                     False),
                           p['angle_emb'], N_LG, 1280)
    ez = _mm_pair(t2a, affa, jax.nn.softplus,
                  p['layers'][0]['edge']['We'], p['layers'][0]['edge']['be'],
                  p['layers'][1]['edge']['We'], p['layers'][1]['edge']['be'],
                  N_LG, 1280)

    for li, lp in enumerate(p['layers']):
        x, mg = _cgcnn_node(lp['node'], x, y, src, dst, dst2d, arange)
        wb_top = lp['Wb'][:64]
        wb_bot = lp['Wb'][64:]
        mbt, stb = _mm([(y, None, None, None, wb_top),
                        (mg, None, None, None, wb_bot)], lp['bb'],
                       N_EDGES, 1280, stats=True)
        ab, cb = _aff_from_stats(stb, N_EDGES, lp['gb'], lp['bbn'])
        y = _res_update(mbt, y, ab, cb, 'sp_in', N_EDGES, 1280)
        y = _cgcnn_edge(lp['edge'], y, ez[li], lsrc, ldst, ldst2d, arange)

    (xf, stf), _ = _cgcnn_node(p['final'], x, y, src, dst, dst2d, arange,
                               upd_stats=True)
    af, cf = _aff_from_stats(stf, N_NODES, p['g_f'], p['b_f'])
    out = _pool_head(xf, af, cf, p['W_fc'], p['b_fc'], N_NODES, 1000)
    return out.reshape(())
